# Initial kernel scaffold; baseline (speedup 1.0000x reference)
import jax
import jax.numpy as jnp
from jax.experimental import pallas as pl


def kernel(x, Key, init_mat, Value, W1, b1, W2, b2):
    raise NotImplementedError("write your pallas kernel here")


# speedup vs baseline: 2.8977x; 2.8977x over previous
import functools

import jax
import jax.numpy as jnp
from jax import lax
from jax.experimental import pallas as pl
from jax.experimental.pallas import tpu as pltpu
from jax.experimental.pallas import tpu_sc as plsc

N_ANCHORS = 2048
D_IN = 3
D_OUT = 256
HID = 1024
NQ = 32768
BQ = 1024
NBLK = NQ // BQ

_SC_NC = 2
_SC_NS = 16
_NW = _SC_NC * _SC_NS
_B_PER_W = NQ // _NW
_CH = 128
_NCHUNK = _B_PER_W // _CH


def _tc_body(x_ref, keyt_ref, key_ref, w1t_ref, b1_ref, w2t_ref, b2_ref,
             im_ref, val_ref, idx_ref, mlp_ref, v_ref):
    xb = x_ref[...]
    kt = keyt_ref[...]
    sqk = jnp.sum(kt * kt, axis=0, keepdims=True)
    sc = sqk - 2.0 * jnp.dot(xb, kt, preferred_element_type=jnp.float32)
    idx = jnp.argmin(sc, axis=1).astype(jnp.int32)
    idx_ref[0, 0, :] = idx
    oh = (lax.broadcasted_iota(jnp.int32, sc.shape, 1) == idx[:, None])
    ohf = oh.astype(jnp.float32)
    keyg = jnp.dot(ohf, key_ref[...], preferred_element_type=jnp.float32)
    rl = xb - keyg
    h1 = jnp.dot(rl, w1t_ref[...], preferred_element_type=jnp.float32) + b1_ref[...]
    hg = 0.5 * h1 * (1.0 + lax.erf(h1 * (2.0 ** -0.5)))
    mlp_ref[...] = jnp.dot(hg, w2t_ref[...], preferred_element_type=jnp.float32) + b2_ref[...]

    @pl.when(pl.program_id(0) == 0)
    def _():
        v_ref[...] = im_ref[...] + val_ref[...]


def _tc_stage(xf, keyt, key, w1t, b1r, w2t, b2r, init_mat, value):
    whole = lambda shape: pl.BlockSpec(shape, lambda i: (0,) * len(shape))
    return pl.pallas_call(
        _tc_body,
        grid=(NBLK,),
        in_specs=[
            pl.BlockSpec((BQ, D_IN), lambda i: (i, 0)),
            whole((D_IN, N_ANCHORS)),
            whole((N_ANCHORS, D_IN)),
            whole((D_IN, HID)),
            whole((1, HID)),
            whole((HID, D_OUT)),
            whole((1, D_OUT)),
            whole((N_ANCHORS, D_OUT)),
            whole((N_ANCHORS, D_OUT)),
        ],
        out_specs=[
            pl.BlockSpec((1, 1, BQ), lambda i: (i, 0, 0)),
            pl.BlockSpec((BQ, D_OUT), lambda i: (i, 0)),
            pl.BlockSpec((N_ANCHORS, D_OUT), lambda i: (0, 0)),
        ],
        out_shape=[
            jax.ShapeDtypeStruct((NBLK, 1, BQ), jnp.int32),
            jax.ShapeDtypeStruct((NQ, D_OUT), jnp.float32),
            jax.ShapeDtypeStruct((N_ANCHORS, D_OUT), jnp.float32),
        ],
    )(xf, keyt, key, w1t, b1r, w2t, b2r, init_mat, value)


@functools.cache
def _sc_stage():
    @functools.partial(
        pl.kernel,
        mesh=plsc.VectorSubcoreMesh(core_axis_name="c", subcore_axis_name="s"),
        out_type=jax.ShapeDtypeStruct((NQ, D_OUT), jnp.float32),
        scratch_types=[
            pltpu.VMEM((_CH,), jnp.int32),
            pltpu.VMEM((_CH, D_OUT), jnp.float32),
            pltpu.VMEM((_CH, D_OUT), jnp.float32),
            pltpu.SemaphoreType.DMA,
        ],
    )
    def _sc_gather_add(v_hbm, idx_hbm, mlp_hbm, out_hbm, idx_v, rows_v, mlp_v, sem):
        wid = lax.axis_index("s") * _SC_NC + lax.axis_index("c")
        base = wid * _B_PER_W

        def chunk(i, carry):
            off = base + i * _CH
            pltpu.sync_copy(idx_hbm.at[pl.ds(off, _CH)], idx_v)
            cp = pltpu.async_copy(v_hbm.at[idx_v], rows_v, sem)
            pltpu.sync_copy(mlp_hbm.at[pl.ds(off, _CH)], mlp_v)
            cp.wait()

            def row(r, c2):
                for j in range(D_OUT // 16):
                    s = pl.ds(j * 16, 16)
                    mlp_v[r, s] = mlp_v[r, s] + rows_v[r, s]
                return c2

            lax.fori_loop(0, _CH, row, 0)
            pltpu.sync_copy(mlp_v, out_hbm.at[pl.ds(off, _CH)])
            return carry

        lax.fori_loop(0, _NCHUNK, chunk, 0)

    return _sc_gather_add


def kernel(x, Key, init_mat, Value, W1, b1, W2, b2):
    bsz, tok, _ = x.shape
    xf = x.reshape(NQ, D_IN)
    idx3, mlp, v = _tc_stage(
        xf, Key.T, Key, W1.T, b1.reshape(1, HID), W2.T, b2.reshape(1, D_OUT),
        init_mat, Value)
    out = _sc_stage()(v, idx3.reshape(NQ), mlp)
    return out.reshape(bsz, tok, D_OUT)


# speedup vs baseline: 3.9407x; 1.3600x over previous
import functools

import jax
import jax.numpy as jnp
from jax import lax
from jax.experimental import pallas as pl
from jax.experimental.pallas import tpu as pltpu
from jax.experimental.pallas import tpu_sc as plsc

N_ANCHORS = 2048
D_IN = 3
D_OUT = 256
HID = 1024
NQ = 32768
BQ = 1024
NBLK = NQ // BQ
KPAD = 128

_SC_NC = 2
_SC_NS = 16
_NW = _SC_NC * _SC_NS
_B_PER_W = NQ // _NW
_CH = 128
_NCHUNK = _B_PER_W // _CH


def _tc1_body(x_ref, keyt_ref, im_ref, val_ref, idx_ref, v_ref):
    xb = x_ref[...]
    kt = keyt_ref[...]
    sqk = jnp.sum(kt * kt, axis=0, keepdims=True)
    sc = sqk - 2.0 * jnp.dot(xb, kt, preferred_element_type=jnp.float32)
    idx_ref[0, 0, :] = jnp.argmin(sc, axis=1).astype(jnp.int32)

    @pl.when(pl.program_id(0) == 0)
    def _():
        v_ref[...] = im_ref[...] + val_ref[...]


def _tc1_stage(xf, keyt, init_mat, value):
    whole = lambda shape: pl.BlockSpec(shape, lambda i: (0,) * len(shape))
    return pl.pallas_call(
        _tc1_body,
        grid=(NBLK,),
        in_specs=[
            pl.BlockSpec((BQ, D_IN), lambda i: (i, 0)),
            whole((D_IN, N_ANCHORS)),
            whole((N_ANCHORS, D_OUT)),
            whole((N_ANCHORS, D_OUT)),
        ],
        out_specs=[
            pl.BlockSpec((1, 1, BQ), lambda i: (i, 0, 0)),
            pl.BlockSpec((N_ANCHORS, D_OUT), lambda i: (0, 0)),
        ],
        out_shape=[
            jax.ShapeDtypeStruct((NBLK, 1, BQ), jnp.int32),
            jax.ShapeDtypeStruct((N_ANCHORS, D_OUT), jnp.float32),
        ],
    )(xf, keyt, init_mat, value)


@functools.cache
def _sc_stage():
    @functools.partial(
        pl.kernel,
        mesh=plsc.VectorSubcoreMesh(core_axis_name="c", subcore_axis_name="s"),
        out_type=[
            jax.ShapeDtypeStruct((NQ, D_OUT), jnp.float32),
            jax.ShapeDtypeStruct((NQ, KPAD), jnp.float32),
        ],
        scratch_types=[
            pltpu.VMEM((_CH,), jnp.int32),
            pltpu.VMEM((_CH, D_OUT), jnp.float32),
            pltpu.VMEM((_CH, KPAD), jnp.float32),
            pltpu.SemaphoreType.DMA,
            pltpu.SemaphoreType.DMA,
        ],
    )
    def _sc_gather(v_hbm, key_hbm, idx_hbm, vg_hbm, kg_hbm,
                   idx_v, rows_v, krows_v, sem, sem2):
        wid = lax.axis_index("s") * _SC_NC + lax.axis_index("c")
        base = wid * _B_PER_W

        def chunk(i, carry):
            off = base + i * _CH
            pltpu.sync_copy(idx_hbm.at[pl.ds(off, _CH)], idx_v)
            cp1 = pltpu.async_copy(v_hbm.at[idx_v], rows_v, sem)
            cp2 = pltpu.async_copy(key_hbm.at[idx_v], krows_v, sem2)
            cp1.wait()
            cp2.wait()
            pltpu.sync_copy(rows_v, vg_hbm.at[pl.ds(off, _CH)])
            pltpu.sync_copy(krows_v, kg_hbm.at[pl.ds(off, _CH)])
            return carry

        lax.fori_loop(0, _NCHUNK, chunk, 0)

    return _sc_gather


def _tc2_body(x_ref, kg_ref, vg_ref, w1t_ref, b1_ref, w2t_ref, b2_ref, out_ref):
    xb = x_ref[...]
    rl = xb - kg_ref[:, :D_IN]
    h1 = jnp.dot(rl, w1t_ref[...], preferred_element_type=jnp.float32) + b1_ref[...]
    hg = 0.5 * h1 * (1.0 + lax.erf(h1 * (2.0 ** -0.5)))
    out_ref[...] = (jnp.dot(hg, w2t_ref[...], preferred_element_type=jnp.float32)
                    + b2_ref[...] + vg_ref[...])


def _tc2_stage(xf, kg, vg, w1t, b1r, w2t, b2r):
    whole = lambda shape: pl.BlockSpec(shape, lambda i: (0,) * len(shape))
    return pl.pallas_call(
        _tc2_body,
        grid=(NBLK,),
        in_specs=[
            pl.BlockSpec((BQ, D_IN), lambda i: (i, 0)),
            pl.BlockSpec((BQ, KPAD), lambda i: (i, 0)),
            pl.BlockSpec((BQ, D_OUT), lambda i: (i, 0)),
            whole((D_IN, HID)),
            whole((1, HID)),
            whole((HID, D_OUT)),
            whole((1, D_OUT)),
        ],
        out_specs=pl.BlockSpec((BQ, D_OUT), lambda i: (i, 0)),
        out_shape=jax.ShapeDtypeStruct((NQ, D_OUT), jnp.float32),
    )(xf, kg, vg, w1t, b1r, w2t, b2r)


def kernel(x, Key, init_mat, Value, W1, b1, W2, b2):
    bsz, tok, _ = x.shape
    xf = x.reshape(NQ, D_IN)
    key16 = jnp.pad(Key, ((0, 0), (0, KPAD - D_IN)))
    idx3, v = _tc1_stage(xf, Key.T, init_mat, Value)
    vg, kg = _sc_stage()(v, key16, idx3.reshape(NQ))
    out = _tc2_stage(xf, kg, vg, W1.T, b1.reshape(1, HID), W2.T,
                     b2.reshape(1, D_OUT))
    return out.reshape(bsz, tok, D_OUT)


# speedup vs baseline: 4.0460x; 1.0267x over previous
import functools

import jax
import jax.numpy as jnp
from jax import lax
from jax.experimental import pallas as pl
from jax.experimental.pallas import tpu as pltpu
from jax.experimental.pallas import tpu_sc as plsc

N_ANCHORS = 2048
D_IN = 3
D_OUT = 256
HID = 1024
NQ = 32768
BQ = 1024
NBLK = NQ // BQ
KG = 4

_SC_NC = 2
_SC_NS = 16
_NW = _SC_NC * _SC_NS
_B_PER_W = NQ // _NW
_CH = 128
_NCHUNK = _B_PER_W // _CH


def _tc1_body(xt_ref, key_ref, im_ref, val_ref, idx_ref, v_ref):
    key = key_ref[...]
    sqk = jnp.sum(key * key, axis=1, keepdims=True)
    sct = sqk - 2.0 * jnp.dot(key, xt_ref[...],
                              preferred_element_type=jnp.float32)
    idx_ref[0, 0, :] = jnp.argmin(sct, axis=0).astype(jnp.int32)

    @pl.when(pl.program_id(0) == 0)
    def _():
        v_ref[...] = im_ref[...] + val_ref[...]


def _tc1_stage(xt, key, init_mat, value):
    whole = lambda shape: pl.BlockSpec(shape, lambda i: (0,) * len(shape))
    return pl.pallas_call(
        _tc1_body,
        grid=(NBLK,),
        in_specs=[
            pl.BlockSpec((D_IN, BQ), lambda i: (0, i)),
            whole((N_ANCHORS, D_IN)),
            whole((N_ANCHORS, D_OUT)),
            whole((N_ANCHORS, D_OUT)),
        ],
        out_specs=[
            pl.BlockSpec((1, 1, BQ), lambda i: (i, 0, 0)),
            pl.BlockSpec((N_ANCHORS, D_OUT), lambda i: (0, 0)),
        ],
        out_shape=[
            jax.ShapeDtypeStruct((NBLK, 1, BQ), jnp.int32),
            jax.ShapeDtypeStruct((N_ANCHORS, D_OUT), jnp.float32),
        ],
    )(xt, key, init_mat, value)


@functools.cache
def _sc_stage():
    @functools.partial(
        pl.kernel,
        mesh=plsc.VectorSubcoreMesh(core_axis_name="c", subcore_axis_name="s"),
        compiler_params=pltpu.CompilerParams(needs_layout_passes=False),
        out_type=[
            jax.ShapeDtypeStruct((NQ, D_OUT), jnp.float32),
            jax.ShapeDtypeStruct((NQ * KG,), jnp.float32),
        ],
        scratch_types=[
            pltpu.VMEM((_CH,), jnp.int32),
            pltpu.VMEM((_CH, D_OUT), jnp.float32),
            pltpu.VMEM((N_ANCHORS * KG,), jnp.float32),
            pltpu.VMEM((_CH * KG,), jnp.float32),
            pltpu.SemaphoreType.DMA,
        ],
    )
    def _sc_gather(v_hbm, key_hbm, idx_hbm, vg_hbm, kg_hbm,
                   idx_v, rows_v, key_v, kg_v, sem):
        wid = lax.axis_index("s") * _SC_NC + lax.axis_index("c")
        base = wid * _B_PER_W
        pltpu.sync_copy(key_hbm, key_v)

        def chunk(i, carry):
            off = base + i * _CH
            pltpu.sync_copy(idx_hbm.at[pl.ds(off, _CH)], idx_v)
            cp1 = pltpu.async_copy(v_hbm.at[idx_v], rows_v, sem)
            lq = lax.iota(jnp.int32, 16)
            for g in range(_CH // 16):
                qv = idx_v[pl.ds(g * 16, 16)] * KG
                dst = (lq + g * 16) * KG
                for c in range(D_IN):
                    vals = plsc.load_gather(key_v, [qv + c])
                    plsc.store_scatter(kg_v, [dst + c], vals)
            cp1.wait()
            pltpu.sync_copy(rows_v, vg_hbm.at[pl.ds(off, _CH)])
            pltpu.sync_copy(kg_v, kg_hbm.at[pl.ds(off * KG, _CH * KG)])
            return carry

        lax.fori_loop(0, _NCHUNK, chunk, 0)

    return _sc_gather


def _tc2_body(x_ref, kg_ref, vg_ref, w1t_ref, b1_ref, w2t_ref, b2_ref, out_ref):
    xb = x_ref[...]
    rl = xb - kg_ref[:, :D_IN]
    h1 = jnp.dot(rl, w1t_ref[...], preferred_element_type=jnp.float32) + b1_ref[...]
    hg = 0.5 * h1 * (1.0 + lax.erf(h1 * (2.0 ** -0.5)))
    out_ref[...] = (jnp.dot(hg, w2t_ref[...], preferred_element_type=jnp.float32)
                    + b2_ref[...] + vg_ref[...])


def _tc2_stage(xf, kg, vg, w1t, b1r, w2t, b2r):
    whole = lambda shape: pl.BlockSpec(shape, lambda i: (0,) * len(shape))
    return pl.pallas_call(
        _tc2_body,
        grid=(NBLK,),
        in_specs=[
            pl.BlockSpec((BQ, D_IN), lambda i: (i, 0)),
            pl.BlockSpec((BQ, KG), lambda i: (i, 0)),
            pl.BlockSpec((BQ, D_OUT), lambda i: (i, 0)),
            whole((D_IN, HID)),
            whole((1, HID)),
            whole((HID, D_OUT)),
            whole((1, D_OUT)),
        ],
        out_specs=pl.BlockSpec((BQ, D_OUT), lambda i: (i, 0)),
        out_shape=jax.ShapeDtypeStruct((NQ, D_OUT), jnp.float32),
    )(xf, kg, vg, w1t, b1r, w2t, b2r)


def kernel(x, Key, init_mat, Value, W1, b1, W2, b2):
    bsz, tok, _ = x.shape
    xf = x.reshape(NQ, D_IN)
    key4 = jnp.pad(Key, ((0, 0), (0, KG - D_IN))).reshape(N_ANCHORS * KG)
    idx3, v = _tc1_stage(xf.T, Key, init_mat, Value)
    vg, kgf = _sc_stage()(v, key4, idx3.reshape(NQ))
    out = _tc2_stage(xf, kgf.reshape(NQ, KG), vg, W1.T, b1.reshape(1, HID),
                     W2.T, b2.reshape(1, D_OUT))
    return out.reshape(bsz, tok, D_OUT)


# speedup vs baseline: 4.5333x; 1.1204x over previous
import functools

import jax
import jax.numpy as jnp
from jax import lax
from jax.experimental import pallas as pl
from jax.experimental.pallas import tpu as pltpu
from jax.experimental.pallas import tpu_sc as plsc

N_ANCHORS = 2048
D_IN = 3
D_OUT = 256
HID = 1024
NQ = 32768
BQ = 1024
NBLK = NQ // BQ
KG = 4

_SC_NC = 2
_SC_NS = 16
_NW = _SC_NC * _SC_NS
_B_PER_W = NQ // _NW
_CH = 128
_NCHUNK = _B_PER_W // _CH


def _tc1_body(x_ref, key_ref, im_ref, val_ref, idx_ref, v_ref):
    key = key_ref[...]
    sqk = jnp.sum(key * key, axis=1, keepdims=True)
    prod = lax.dot_general(key, x_ref[...], (((1,), (1,)), ((), ())),
                           preferred_element_type=jnp.float32)
    sct = sqk - 2.0 * prod
    idx_ref[0, 0, :] = jnp.argmin(sct, axis=0).astype(jnp.int32)

    @pl.when(pl.program_id(0) == 0)
    def _():
        v_ref[...] = im_ref[...] + val_ref[...]


def _tc1_stage(xf, key, init_mat, value):
    whole = lambda shape: pl.BlockSpec(shape, lambda i: (0,) * len(shape))
    return pl.pallas_call(
        _tc1_body,
        grid=(NBLK,),
        in_specs=[
            pl.BlockSpec((BQ, D_IN), lambda i: (i, 0)),
            whole((N_ANCHORS, D_IN)),
            whole((N_ANCHORS, D_OUT)),
            whole((N_ANCHORS, D_OUT)),
        ],
        out_specs=[
            pl.BlockSpec((1, 1, BQ), lambda i: (i, 0, 0)),
            pl.BlockSpec((N_ANCHORS, D_OUT), lambda i: (0, 0)),
        ],
        out_shape=[
            jax.ShapeDtypeStruct((NBLK, 1, BQ), jnp.int32),
            jax.ShapeDtypeStruct((N_ANCHORS, D_OUT), jnp.float32),
        ],
    )(xf, key, init_mat, value)


@functools.cache
def _sc_stage():
    @functools.partial(
        pl.kernel,
        mesh=plsc.VectorSubcoreMesh(core_axis_name="c", subcore_axis_name="s"),
        compiler_params=pltpu.CompilerParams(needs_layout_passes=False),
        out_type=[
            jax.ShapeDtypeStruct((NQ, D_OUT), jnp.float32),
            jax.ShapeDtypeStruct((NQ, KG), jnp.float32),
        ],
        scratch_types=[
            pltpu.VMEM((_CH,), jnp.int32),
            pltpu.VMEM((_CH, D_OUT), jnp.float32),
            pltpu.VMEM((N_ANCHORS * KG,), jnp.float32),
            pltpu.VMEM((_CH, KG), jnp.float32),
            pltpu.SemaphoreType.DMA,
        ],
    )
    def _sc_gather(v_hbm, key_hbm, idx_hbm, vg_hbm, kg_hbm,
                   idx_v, rows_v, key_v, kg_v, sem):
        wid = lax.axis_index("s") * _SC_NC + lax.axis_index("c")
        base = wid * _B_PER_W
        pltpu.sync_copy(key_hbm, key_v)

        def chunk(i, carry):
            off = base + i * _CH
            pltpu.sync_copy(idx_hbm.at[pl.ds(off, _CH)], idx_v)
            cp1 = pltpu.async_copy(v_hbm.at[idx_v], rows_v, sem)
            lq = lax.iota(jnp.int32, 16)
            for g in range(_CH // 16):
                qv = idx_v[pl.ds(g * 16, 16)] * KG
                rows = lq + g * 16
                for c in range(D_IN):
                    vals = plsc.load_gather(key_v, [qv + c])
                    plsc.store_scatter(kg_v, [rows, lq * 0 + c], vals)
            cp1.wait()
            pltpu.sync_copy(rows_v, vg_hbm.at[pl.ds(off, _CH)])
            pltpu.sync_copy(kg_v, kg_hbm.at[pl.ds(off, _CH)])
            return carry

        lax.fori_loop(0, _NCHUNK, chunk, 0)

    return _sc_gather


def _tc2_body(x_ref, kg_ref, vg_ref, w1t_ref, b1_ref, w2t_ref, b2_ref, out_ref):
    xb = x_ref[...]
    rl = xb - kg_ref[:, :D_IN]
    h1 = jnp.dot(rl, w1t_ref[...], preferred_element_type=jnp.float32) + b1_ref[...]
    hg = 0.5 * h1 * (1.0 + lax.erf(h1 * (2.0 ** -0.5)))
    out_ref[0, :, :] = (jnp.dot(hg, w2t_ref[...], preferred_element_type=jnp.float32)
                        + b2_ref[...] + vg_ref[...])


_TPB = 8192 // BQ


def _tc2_stage(xf, kg, vg, w1t, b1r, w2t, b2r):
    whole = lambda shape: pl.BlockSpec(shape, lambda i: (0,) * len(shape))
    return pl.pallas_call(
        _tc2_body,
        grid=(NBLK,),
        in_specs=[
            pl.BlockSpec((BQ, D_IN), lambda i: (i, 0)),
            pl.BlockSpec((BQ, KG), lambda i: (i, 0)),
            pl.BlockSpec((BQ, D_OUT), lambda i: (i, 0)),
            whole((D_IN, HID)),
            whole((1, HID)),
            whole((HID, D_OUT)),
            whole((1, D_OUT)),
        ],
        out_specs=pl.BlockSpec((1, BQ, D_OUT), lambda i: (i // _TPB, i % _TPB, 0)),
        out_shape=jax.ShapeDtypeStruct((4, 8192, D_OUT), jnp.float32),
    )(xf, kg, vg, w1t, b1r, w2t, b2r)


def kernel(x, Key, init_mat, Value, W1, b1, W2, b2):
    bsz, tok, _ = x.shape
    xf = x.reshape(NQ, D_IN)
    key4 = jnp.pad(Key, ((0, 0), (0, KG - D_IN))).reshape(N_ANCHORS * KG)
    idx3, v = _tc1_stage(xf, Key, init_mat, Value)
    vg, kg = _sc_stage()(v, key4, idx3.reshape(NQ))
    return _tc2_stage(xf, kg, vg, W1.T, b1.reshape(1, HID), W2.T,
                      b2.reshape(1, D_OUT))


# speedup vs baseline: 4.6758x; 1.0314x over previous
import functools

import jax
import jax.numpy as jnp
from jax import lax
from jax.experimental import pallas as pl
from jax.experimental.pallas import tpu as pltpu
from jax.experimental.pallas import tpu_sc as plsc

N_ANCHORS = 2048
D_IN = 3
D_OUT = 256
HID = 1024
NQ = 32768
BQ = 1024
NBLK = NQ // BQ
KG = 4

_SC_NC = 2
_SC_NS = 16
_NW = _SC_NC * _SC_NS
_B_PER_W = NQ // _NW
_CH = 128
_NCHUNK = _B_PER_W // _CH


def _tc1_body(x_ref, key_ref, im_ref, val_ref, idx_ref, v_ref):
    key = key_ref[...]
    sqk = jnp.sum(key * key, axis=1, keepdims=True)
    prod = lax.dot_general(key, x_ref[...], (((1,), (1,)), ((), ())),
                           preferred_element_type=jnp.float32)
    sct = sqk - 2.0 * prod
    idx_ref[...] = jnp.argmin(sct, axis=0).astype(jnp.int32)

    @pl.when(pl.program_id(0) == 0)
    def _():
        v_ref[...] = im_ref[...] + val_ref[...]


def _tc1_stage(xf, key, init_mat, value):
    whole = lambda shape: pl.BlockSpec(shape, lambda i: (0,) * len(shape))
    return pl.pallas_call(
        _tc1_body,
        grid=(NBLK,),
        in_specs=[
            pl.BlockSpec((BQ, D_IN), lambda i: (i, 0)),
            whole((N_ANCHORS, D_IN)),
            whole((N_ANCHORS, D_OUT)),
            whole((N_ANCHORS, D_OUT)),
        ],
        out_specs=[
            pl.BlockSpec((BQ,), lambda i: (i,)),
            pl.BlockSpec((N_ANCHORS, D_OUT), lambda i: (0, 0)),
        ],
        out_shape=[
            jax.ShapeDtypeStruct((NQ,), jnp.int32),
            jax.ShapeDtypeStruct((N_ANCHORS, D_OUT), jnp.float32),
        ],
    )(xf, key, init_mat, value)


@functools.cache
def _sc_stage():
    @functools.partial(
        pl.kernel,
        mesh=plsc.VectorSubcoreMesh(core_axis_name="c", subcore_axis_name="s"),
        compiler_params=pltpu.CompilerParams(needs_layout_passes=False),
        out_type=[
            jax.ShapeDtypeStruct((NQ, D_OUT), jnp.float32),
            jax.ShapeDtypeStruct((NQ, KG), jnp.float32),
        ],
        scratch_types=[
            pltpu.VMEM((_CH,), jnp.int32),
            pltpu.VMEM((_CH,), jnp.int32),
            pltpu.VMEM((_CH, D_OUT), jnp.float32),
            pltpu.VMEM((_CH, D_OUT), jnp.float32),
            pltpu.VMEM((N_ANCHORS * KG,), jnp.float32),
            pltpu.VMEM((_CH, KG), jnp.float32),
            pltpu.VMEM((_CH, KG), jnp.float32),
            pltpu.SemaphoreType.DMA,
            pltpu.SemaphoreType.DMA,
            pltpu.SemaphoreType.DMA,
            pltpu.SemaphoreType.DMA,
        ],
    )
    def _sc_gather(v_hbm, key_hbm, idx_hbm, vg_hbm, kg_hbm,
                   idx0, idx1, rows0, rows1, key_v, kg0, kg1,
                   sg0, sg1, sw0, sw1):
        idx_v = [idx0, idx1]
        rows_v = [rows0, rows1]
        kg_v = [kg0, kg1]
        sg = [sg0, sg1]
        sw = [sw0, sw1]
        wid = lax.axis_index("s") * _SC_NC + lax.axis_index("c")
        base = wid * _B_PER_W
        pltpu.sync_copy(key_hbm, key_v)
        lq = lax.iota(jnp.int32, 16)

        gth = [None, None]
        wbk = [[], []]

        def start(i):
            b = i % 2
            off = base + i * _CH
            pltpu.sync_copy(idx_hbm.at[pl.ds(off, _CH)], idx_v[b])
            gth[b] = pltpu.async_copy(v_hbm.at[idx_v[b]], rows_v[b], sg[b])

        start(0)
        for i in range(_NCHUNK):
            b = i % 2
            nb = (i + 1) % 2
            if i + 1 < _NCHUNK:
                for h in wbk[nb]:
                    h.wait()
                wbk[nb] = []
                start(i + 1)
            for g in range(_CH // 16):
                qv = idx_v[b][pl.ds(g * 16, 16)] * KG
                rows = lq + g * 16
                for c in range(D_IN):
                    vals = plsc.load_gather(key_v, [qv + c])
                    plsc.store_scatter(kg_v[b], [rows, lq * 0 + c], vals)
            gth[b].wait()
            off = base + i * _CH
            wbk[b] = [
                pltpu.async_copy(rows_v[b], vg_hbm.at[pl.ds(off, _CH)], sw[b]),
                pltpu.async_copy(kg_v[b], kg_hbm.at[pl.ds(off, _CH)], sw[b]),
            ]
        for lst in wbk:
            for h in lst:
                h.wait()

    return _sc_gather


def _tc2_body(x_ref, kg_ref, vg_ref, w1t_ref, b1_ref, w2t_ref, b2_ref, out_ref):
    xb = x_ref[...]
    rl = xb - kg_ref[:, :D_IN]
    h1 = jnp.dot(rl, w1t_ref[...], preferred_element_type=jnp.float32) + b1_ref[...]
    hg = 0.5 * h1 * (1.0 + lax.erf(h1 * (2.0 ** -0.5)))
    out_ref[0, :, :] = (jnp.dot(hg, w2t_ref[...], preferred_element_type=jnp.float32)
                        + b2_ref[...] + vg_ref[...])


_TPB = 8192 // BQ


def _tc2_stage(xf, kg, vg, w1t, b1r, w2t, b2r):
    whole = lambda shape: pl.BlockSpec(shape, lambda i: (0,) * len(shape))
    return pl.pallas_call(
        _tc2_body,
        grid=(NBLK,),
        in_specs=[
            pl.BlockSpec((BQ, D_IN), lambda i: (i, 0)),
            pl.BlockSpec((BQ, KG), lambda i: (i, 0)),
            pl.BlockSpec((BQ, D_OUT), lambda i: (i, 0)),
            whole((D_IN, HID)),
            whole((1, HID)),
            whole((HID, D_OUT)),
            whole((1, D_OUT)),
        ],
        out_specs=pl.BlockSpec((1, BQ, D_OUT), lambda i: (i // _TPB, i % _TPB, 0)),
        out_shape=jax.ShapeDtypeStruct((4, 8192, D_OUT), jnp.float32),
    )(xf, kg, vg, w1t, b1r, w2t, b2r)


def kernel(x, Key, init_mat, Value, W1, b1, W2, b2):
    bsz, tok, _ = x.shape
    xf = x.reshape(NQ, D_IN)
    key4 = jnp.pad(Key, ((0, 0), (0, KG - D_IN))).reshape(N_ANCHORS * KG)
    idx, v = _tc1_stage(xf, Key, init_mat, Value)
    vg, kg = _sc_stage()(v, key4, idx)
    return _tc2_stage(xf, kg, vg, W1.T, b1.reshape(1, HID), W2.T,
                      b2.reshape(1, D_OUT))
